# Initial kernel scaffold; baseline (speedup 1.0000x reference)
import jax
import jax.numpy as jnp
from jax.experimental import pallas as pl


def kernel(inputs_0, inputs_1, inputs_2, inputs_3, W0, W1, W2, W3):
    raise NotImplementedError("write your pallas kernel here")


# speedup vs baseline: 2.6397x; 2.6397x over previous
import functools

import jax
import jax.numpy as jnp
from jax import lax
from jax.experimental import pallas as pl
from jax.experimental.pallas import tpu as pltpu
from jax.experimental.pallas import tpu_sc as plsc

_EMB = (64, 32, 32, 16)
_OFF = (0, 64, 96, 128)
_DTOT = 144
_B, _L = 4096, 50
_N = _B * _L
_NW = 32
_ROWS_PER_W = _N // _NW
_CHUNK = 256
_IDXW = 128
_CROWS = _CHUNK // _IDXW
_NCHUNK = _ROWS_PER_W // _CHUNK

_mesh = plsc.VectorSubcoreMesh(core_axis_name="c", subcore_axis_name="s")


@functools.partial(
    pl.kernel,
    mesh=_mesh,
    out_type=jax.ShapeDtypeStruct((_N, _DTOT), jnp.float32),
    scratch_types=(
        [pltpu.VMEM((_CROWS, _IDXW), jnp.int32) for _ in range(4)]
        + [pltpu.VMEM((_CHUNK, e), jnp.float32) for e in _EMB]
        + [pltpu.SemaphoreType.DMA]
    ),
    compiler_params=pltpu.CompilerParams(use_tc_tiling_on_sc=False),
)
def _emb_kernel(i0, i1, i2, i3, w0, w1, w2, w3, out,
                x0, x1, x2, x3, g0, g1, g2, g3, sem):
    wid = lax.axis_index("s") * 2 + lax.axis_index("c")
    idx_hbm = (i0, i1, i2, i3)
    tables = (w0, w1, w2, w3)
    idx_v = (x0, x1, x2, x3)
    gat_v = (g0, g1, g2, g3)

    def body(ci, carry):
        base = wid * _ROWS_PER_W + ci * _CHUNK
        irow = wid * (_ROWS_PER_W // _IDXW) + ci * _CROWS
        for t in range(4):
            pltpu.sync_copy(idx_hbm[t].at[pl.ds(irow, _CROWS)], idx_v[t])
        waits = []
        for t in range(4):
            for j in range(_CROWS):
                waits.append(pltpu.async_copy(
                    tables[t].at[idx_v[t].at[j]],
                    gat_v[t].at[pl.ds(j * _IDXW, _IDXW)],
                    sem))
        for w in waits:
            w.wait()
        for t in range(4):
            pltpu.sync_copy(
                gat_v[t],
                out.at[pl.ds(base, _CHUNK), pl.ds(_OFF[t], _EMB[t])])
        return carry

    lax.fori_loop(0, _NCHUNK, body, 0)


def kernel(inputs_0, inputs_1, inputs_2, inputs_3, W0, W1, W2, W3):
    idxs = [x.astype(jnp.int32).reshape(_N // _IDXW, _IDXW)
            for x in (inputs_0, inputs_1, inputs_2, inputs_3)]
    out = _emb_kernel(*idxs, W0, W1, W2, W3)
    return out.reshape(_B, _L, _DTOT)


# speedup vs baseline: 2.7785x; 1.0526x over previous
import functools

import jax
import jax.numpy as jnp
from jax import lax
from jax.experimental import pallas as pl
from jax.experimental.pallas import tpu as pltpu
from jax.experimental.pallas import tpu_sc as plsc

_EMB = (64, 32, 32, 16)
_OFF = (0, 64, 96, 128)
_DTOT = 144
_B, _L = 4096, 50
_N = _B * _L
_NW = 32
_ROWS_PER_W = _N // _NW
_IDXW = 128
_CHUNK = 128
_NCHUNK = _ROWS_PER_W // _CHUNK
_DEPTH = 5
_NOUTER = _NCHUNK // _DEPTH

_mesh = plsc.VectorSubcoreMesh(core_axis_name="c", subcore_axis_name="s")


@functools.partial(
    pl.kernel,
    mesh=_mesh,
    out_type=jax.ShapeDtypeStruct((_N, _DTOT), jnp.float32),
    scratch_types=(
        [pltpu.VMEM((_NCHUNK, _IDXW), jnp.int32) for _ in range(4)]
        + [[pltpu.VMEM((_CHUNK, e), jnp.float32) for e in _EMB]
           for _ in range(_DEPTH)]
        + [[pltpu.SemaphoreType.DMA for _ in range(_DEPTH)],
           pltpu.SemaphoreType.DMA]
    ),
    compiler_params=pltpu.CompilerParams(use_tc_tiling_on_sc=False),
)
def _emb_kernel(i0, i1, i2, i3, w0, w1, w2, w3, out,
                x0, x1, x2, x3, slot0, slot1, slot2, slot3, slot4,
                gat_sems, out_sem):
    wid = lax.axis_index("s") * 2 + lax.axis_index("c")
    idx_hbm = (i0, i1, i2, i3)
    tables = (w0, w1, w2, w3)
    idx_v = (x0, x1, x2, x3)
    slots = (slot0, slot1, slot2, slot3, slot4)

    for t in range(4):
        pltpu.sync_copy(idx_hbm[t].at[pl.ds(wid * _NCHUNK, _NCHUNK)],
                        idx_v[t])

    def fire_gathers(ci, j):
        for t in range(4):
            pltpu.async_copy(tables[t].at[idx_v[t].at[ci]],
                             slots[j][t], gat_sems[j])

    def wait_gathers(j):
        for t in range(4):
            pltpu.make_async_copy(tables[t].at[pl.ds(0, _CHUNK)],
                                  slots[j][t], gat_sems[j]).wait()

    for j in range(_DEPTH):
        fire_gathers(j, j)

    def body(k, carry):
        for j in range(_DEPTH):
            ci = k * _DEPTH + j
            base = wid * _ROWS_PER_W + ci * _CHUNK
            wait_gathers(j)
            writes = [
                pltpu.async_copy(
                    slots[j][t],
                    out.at[pl.ds(base, _CHUNK), pl.ds(_OFF[t], _EMB[t])],
                    out_sem)
                for t in range(4)]
            for w in writes:
                w.wait()

            @pl.when(k < _NOUTER - 1)
            def _():
                fire_gathers(ci + _DEPTH, j)

        return carry

    lax.fori_loop(0, _NOUTER, body, 0)


def kernel(inputs_0, inputs_1, inputs_2, inputs_3, W0, W1, W2, W3):
    idxs = [x.astype(jnp.int32).reshape(_N // _IDXW, _IDXW)
            for x in (inputs_0, inputs_1, inputs_2, inputs_3)]
    out = _emb_kernel(*idxs, W0, W1, W2, W3)
    return out.reshape(_B, _L, _DTOT)
